# baseline (speedup 1.0000x reference)
import functools

import jax
import jax.numpy as jnp
from jax import lax
from jax.experimental import pallas as pl
from jax.experimental.pallas import tpu as pltpu
from jax.experimental.pallas import tpu_sc as plsc

_ROW = 768
_CH = 32
_L = 16


def _build_sc_call(B, NC, NS, num_kp, d_dist, d_state):
    NW = NC * NS
    b_per_w = B // NW
    n_ch = b_per_w // _CH
    d_out = d_dist + d_state
    mesh = plsc.VectorSubcoreMesh(core_axis_name="c", subcore_axis_name="s")

    @functools.partial(
        pl.kernel,
        mesh=mesh,
        out_type=jax.ShapeDtypeStruct((B, _ROW), jnp.float32),
        scratch_types=[
            pltpu.VMEM((num_kp * d_dist,), jnp.float32),
            pltpu.VMEM((2 * d_state,), jnp.float32),
            pltpu.VMEM((2, _ROW), jnp.float32),
            pltpu.VMEM((b_per_w,), jnp.int32),
            pltpu.VMEM((n_ch, _CH), jnp.int32),
            pltpu.VMEM((4, _CH, _ROW), jnp.float32),
        ] + [pltpu.SemaphoreType.DMA] * 8,
    )
    def sc_gather(dist_hbm, state_hbm, idx_hbm, out_hbm,
                  dist_v, state_v, fused_v, idx_v, idx2_v, rows_v, *sems):
        wid = lax.axis_index("s") * NC + lax.axis_index("c")
        base = wid * b_per_w

        pltpu.sync_copy(dist_hbm, dist_v)
        pltpu.sync_copy(state_hbm, state_v)
        pltpu.sync_copy(idx_hbm.at[pl.ds(base, b_per_w)], idx_v)

        for g in range(2):
            for k in range(num_kp):
                col = k * d_out
                for j in range(d_dist // _L):
                    fused_v[g, pl.ds(col + j * _L, _L)] = (
                        dist_v[pl.ds(k * d_dist + j * _L, _L)])
                for j in range(d_state // _L):
                    fused_v[g, pl.ds(col + d_dist + j * _L, _L)] = (
                        state_v[pl.ds(g * d_state + j * _L, _L)])
        rep = base + b_per_w - 2
        pltpu.sync_copy(fused_v, out_hbm.at[pl.ds(rep, 2)])

        off = jnp.broadcast_to(rep, (_L,)).astype(jnp.int32)
        for c in range(n_ch):
            for j in range(_CH // _L):
                idx2_v[c, pl.ds(j * _L, _L)] = (
                    idx_v[pl.ds(c * _CH + j * _L, _L)] + off)

        NB = 4
        gsem = sems[:NB]
        ssem = sems[NB:]
        gat = [None] * NB
        sca = [None] * NB
        for c in range(min(NB, n_ch)):
            gat[c] = pltpu.async_copy(
                out_hbm.at[idx2_v.at[c]], rows_v.at[c], gsem[c])
        for c in range(n_ch):
            p = c % NB
            gat[p].wait()
            sca[p] = pltpu.async_copy(
                rows_v.at[p], out_hbm.at[pl.ds(base + c * _CH, _CH)], ssem[p])
            if c + NB < n_ch:
                sca[p].wait()
                gat[p] = pltpu.async_copy(
                    out_hbm.at[idx2_v.at[c + NB]], rows_v.at[p], gsem[p])
                sca[p] = None
        for p in range(NB):
            if sca[p] is not None:
                sca[p].wait()

    return sc_gather


def kernel(grip_state, distinction_table, state_table):
    B = grip_state.shape[0]
    num_kp, d_dist = distinction_table.shape
    d_state = state_table.shape[-1]
    info = plsc.get_sparse_core_info()
    NC, NS = info.num_cores, info.num_subcores

    out = _build_sc_call(B, NC, NS, num_kp, d_dist, d_state)(
        distinction_table.reshape(-1),
        state_table.reshape(-1),
        grip_state.astype(jnp.int32))
    return out.reshape(B, num_kp, d_dist + d_state)


# speedup vs baseline: 1.1173x; 1.1173x over previous
import functools

import jax
import jax.numpy as jnp
from jax import lax
from jax.experimental import pallas as pl
from jax.experimental.pallas import tpu as pltpu
from jax.experimental.pallas import tpu_sc as plsc

_CH = 64
_NB = 4
_L = 16


def _build_sc_call(B, NC, NS, num_kp, d_dist, d_state):
    NW = NC * NS
    b_per_w = B // NW
    n_ch = b_per_w // _CH
    d_out = d_dist + d_state
    mesh = plsc.VectorSubcoreMesh(core_axis_name="c", subcore_axis_name="s")

    @functools.partial(
        pl.kernel,
        mesh=mesh,
        out_type=(
            jax.ShapeDtypeStruct((num_kp, B, d_out), jnp.float32),
            jax.ShapeDtypeStruct((NW * 16, d_out), jnp.float32),
        ),
        scratch_types=[
            pltpu.VMEM((num_kp * d_dist,), jnp.float32),
            pltpu.VMEM((2 * d_state,), jnp.float32),
            pltpu.VMEM((16, d_out), jnp.float32),
            pltpu.VMEM((b_per_w,), jnp.int32),
            pltpu.VMEM((num_kp, n_ch, _CH), jnp.int32),
            pltpu.VMEM((_NB, _CH, d_out), jnp.float32),
        ] + [pltpu.SemaphoreType.DMA] * (2 * _NB),
    )
    def sc_gather(dist_hbm, state_hbm, idx_hbm, out_hbm, table_hbm,
                  dist_v, state_v, fused_v, idx_v, idx2_v, rows_v, *sems):
        wid = lax.axis_index("s") * NC + lax.axis_index("c")
        base = wid * b_per_w

        pltpu.sync_copy(dist_hbm, dist_v)
        pltpu.sync_copy(state_hbm, state_v)
        pltpu.sync_copy(idx_hbm.at[pl.ds(base, b_per_w)], idx_v)

        for k in range(num_kp):
            for g in range(2):
                row = 2 * k + g
                for j in range(d_dist // _L):
                    fused_v[row, pl.ds(j * _L, _L)] = (
                        dist_v[pl.ds(k * d_dist + j * _L, _L)])
                for j in range(d_state // _L):
                    fused_v[row, pl.ds(d_dist + j * _L, _L)] = (
                        state_v[pl.ds(g * d_state + j * _L, _L)])
        pltpu.sync_copy(fused_v, table_hbm.at[pl.ds(16 * wid, 16)])

        for k in range(num_kp):
            off = jnp.broadcast_to(16 * wid + 2 * k,
                                   (_L,)).astype(jnp.int32)
            for c in range(n_ch):
                for j in range(_CH // _L):
                    idx2_v[k, c, pl.ds(j * _L, _L)] = (
                        idx_v[pl.ds(c * _CH + j * _L, _L)] + off)

        tiles = [(k, c) for k in range(num_kp) for c in range(n_ch)]
        gsem = sems[:_NB]
        ssem = sems[_NB:]
        gat = [None] * _NB
        sca = [None] * _NB
        for t in range(min(_NB, len(tiles))):
            k, c = tiles[t]
            gat[t] = pltpu.async_copy(
                table_hbm.at[idx2_v.at[k, c]], rows_v.at[t], gsem[t])
        for t in range(len(tiles)):
            p = t % _NB
            k, c = tiles[t]
            gat[p].wait()
            sca[p] = pltpu.async_copy(
                rows_v.at[p],
                out_hbm.at[k, pl.ds(base + c * _CH, _CH)], ssem[p])
            if t + _NB < len(tiles):
                kn, cn = tiles[t + _NB]
                sca[p].wait()
                gat[p] = pltpu.async_copy(
                    table_hbm.at[idx2_v.at[kn, cn]], rows_v.at[p], gsem[p])
                sca[p] = None
        for p in range(_NB):
            if sca[p] is not None:
                sca[p].wait()

    return sc_gather


def kernel(grip_state, distinction_table, state_table):
    B = grip_state.shape[0]
    num_kp, d_dist = distinction_table.shape
    d_state = state_table.shape[-1]
    info = plsc.get_sparse_core_info()
    NC, NS = info.num_cores, info.num_subcores

    out, _ = _build_sc_call(B, NC, NS, num_kp, d_dist, d_state)(
        distinction_table.reshape(-1),
        state_table.reshape(-1),
        grip_state.astype(jnp.int32))
    return jnp.transpose(out, (1, 0, 2))


# speedup vs baseline: 1.1444x; 1.0243x over previous
import functools

import jax
import jax.numpy as jnp
from jax import lax
from jax.experimental import pallas as pl
from jax.experimental.pallas import tpu as pltpu
from jax.experimental.pallas import tpu_sc as plsc

_CH = 64
_NB = 8
_L = 16


def _build_sc_call(B, NC, NS, num_kp, d_dist, d_state):
    NW = NC * NS
    b_per_w = B // NW
    n_ch = b_per_w // _CH
    d_out = d_dist + d_state
    mesh = plsc.VectorSubcoreMesh(core_axis_name="c", subcore_axis_name="s")

    @functools.partial(
        pl.kernel,
        mesh=mesh,
        out_type=(
            jax.ShapeDtypeStruct((num_kp, B, d_out), jnp.float32),
            jax.ShapeDtypeStruct((NW * 16, d_out), jnp.float32),
        ),
        scratch_types=[
            pltpu.VMEM((num_kp * d_dist,), jnp.float32),
            pltpu.VMEM((2 * d_state,), jnp.float32),
            pltpu.VMEM((16, d_out), jnp.float32),
            pltpu.VMEM((b_per_w,), jnp.int32),
            pltpu.VMEM((num_kp, n_ch, _CH), jnp.int32),
            pltpu.VMEM((_NB, _CH, d_out), jnp.float32),
        ] + [pltpu.SemaphoreType.DMA] * (2 * _NB),
    )
    def sc_gather(dist_hbm, state_hbm, idx_hbm, out_hbm, table_hbm,
                  dist_v, state_v, fused_v, idx_v, idx2_v, rows_v, *sems):
        wid = lax.axis_index("s") * NC + lax.axis_index("c")
        base = wid * b_per_w

        pltpu.sync_copy(dist_hbm, dist_v)
        pltpu.sync_copy(state_hbm, state_v)
        pltpu.sync_copy(idx_hbm.at[pl.ds(base, b_per_w)], idx_v)

        for k in range(num_kp):
            for g in range(2):
                row = 2 * k + g
                for j in range(d_dist // _L):
                    fused_v[row, pl.ds(j * _L, _L)] = (
                        dist_v[pl.ds(k * d_dist + j * _L, _L)])
                for j in range(d_state // _L):
                    fused_v[row, pl.ds(d_dist + j * _L, _L)] = (
                        state_v[pl.ds(g * d_state + j * _L, _L)])
        pltpu.sync_copy(fused_v, table_hbm.at[pl.ds(16 * wid, 16)])

        for k in range(num_kp):
            off = jnp.broadcast_to(16 * wid + 2 * k,
                                   (_L,)).astype(jnp.int32)
            for c in range(n_ch):
                for j in range(_CH // _L):
                    idx2_v[k, c, pl.ds(j * _L, _L)] = (
                        idx_v[pl.ds(c * _CH + j * _L, _L)] + off)

        tiles = [(k, c) for k in range(num_kp) for c in range(n_ch)]
        gsem = sems[:_NB]
        ssem = sems[_NB:]
        gat = [None] * _NB
        sca = [None] * _NB
        for t in range(min(_NB, len(tiles))):
            k, c = tiles[t]
            gat[t] = pltpu.async_copy(
                table_hbm.at[idx2_v.at[k, c]], rows_v.at[t], gsem[t])
        for t in range(len(tiles)):
            p = t % _NB
            k, c = tiles[t]
            gat[p].wait()
            sca[p] = pltpu.async_copy(
                rows_v.at[p],
                out_hbm.at[k, pl.ds(base + c * _CH, _CH)], ssem[p])
            if t + _NB < len(tiles):
                kn, cn = tiles[t + _NB]
                sca[p].wait()
                gat[p] = pltpu.async_copy(
                    table_hbm.at[idx2_v.at[kn, cn]], rows_v.at[p], gsem[p])
                sca[p] = None
        for p in range(_NB):
            if sca[p] is not None:
                sca[p].wait()

    return sc_gather


def kernel(grip_state, distinction_table, state_table):
    B = grip_state.shape[0]
    num_kp, d_dist = distinction_table.shape
    d_state = state_table.shape[-1]
    info = plsc.get_sparse_core_info()
    NC, NS = info.num_cores, info.num_subcores

    out, _ = _build_sc_call(B, NC, NS, num_kp, d_dist, d_state)(
        distinction_table.reshape(-1),
        state_table.reshape(-1),
        grip_state.astype(jnp.int32))
    return jnp.transpose(out, (1, 0, 2))


# speedup vs baseline: 2.4997x; 2.1842x over previous
import functools

import jax
import jax.numpy as jnp
from jax import lax
from jax.experimental import pallas as pl
from jax.experimental.pallas import tpu as pltpu
from jax.experimental.pallas import tpu_sc as plsc

_CHT = 16
_NB = 4
_L = 16


def _build_sc_call(B, NC, NS, num_kp, d_dist, d_state):
    NW = NC * NS
    b_per_w = B // NW
    t_per_w = b_per_w // 8
    n_ch = t_per_w // _CHT
    d_out = d_dist + d_state
    nj = d_dist // _L
    mesh = plsc.VectorSubcoreMesh(core_axis_name="c", subcore_axis_name="s")

    @functools.partial(
        pl.kernel,
        mesh=mesh,
        out_type=(
            jax.ShapeDtypeStruct((num_kp, B // 8, 8, d_out), jnp.float32),
            jax.ShapeDtypeStruct((NC, num_kp, 256, 8, d_out), jnp.float32),
        ),
        scratch_types=[
            pltpu.VMEM((num_kp * d_dist,), jnp.float32),
            pltpu.VMEM((2 * d_state,), jnp.float32),
            pltpu.VMEM((NS, 8, d_out), jnp.float32),
            pltpu.VMEM((t_per_w,), jnp.int32),
            pltpu.VMEM((n_ch, _CHT), jnp.int32),
            pltpu.VMEM((_NB, _CHT, 8, d_out), jnp.float32),
        ] + [pltpu.SemaphoreType.DMA] * (2 * _NB),
    )
    def sc_gather(dist_hbm, state_hbm, oct_hbm, out_hbm, table_hbm,
                  dist_v, state_v, stage_v, oct_v, idx_v, rows_v, *sems):
        cid = lax.axis_index("c")
        sid = lax.axis_index("s")
        wid = sid * NC + cid
        tbase = wid * t_per_w

        pltpu.sync_copy(dist_hbm, dist_v)
        pltpu.sync_copy(state_hbm, state_v)
        pltpu.sync_copy(oct_hbm.at[pl.ds(tbase, t_per_w)], oct_v)
        for c in range(n_ch):
            idx_v[c, :] = oct_v[pl.ds(c * _CHT, _L)]

        for k in range(num_kp):
            d = [dist_v[pl.ds(k * d_dist + j * _L, _L)] for j in range(nj)]
            s0 = [state_v[pl.ds(j * _L, _L)] for j in range(nj)]
            s1 = [state_v[pl.ds(d_state + j * _L, _L)] for j in range(nj)]
            rlo = []
            for i in range(4):
                bit = jnp.broadcast_to(
                    ((sid >> i) & 1).astype(jnp.float32), (_L,))
                rlo.append([s0[j] + bit * (s1[j] - s0[j])
                            for j in range(nj)])
            for hi in range(16):
                for i in range(8):
                    if i < 4:
                        right = rlo[i]
                    else:
                        right = s1 if (hi >> (i - 4)) & 1 else s0
                    for j in range(nj):
                        stage_v[hi, i, pl.ds(j * _L, _L)] = d[j]
                        stage_v[hi, i, pl.ds(d_dist + j * _L, _L)] = right[j]
            pltpu.sync_copy(
                stage_v, table_hbm.at[cid, k, pl.ds(16 * sid, NS)])
        plsc.subcore_barrier()

        work = [(k, c) for k in range(num_kp) for c in range(n_ch)]
        gsem = sems[:_NB]
        ssem = sems[_NB:]
        gat = [None] * _NB
        sca = [None] * _NB
        for t in range(min(_NB, len(work))):
            k, c = work[t]
            gat[t] = pltpu.async_copy(
                table_hbm.at[cid, k].at[idx_v.at[c]], rows_v.at[t], gsem[t])
        for t in range(len(work)):
            p = t % _NB
            k, c = work[t]
            gat[p].wait()
            sca[p] = pltpu.async_copy(
                rows_v.at[p],
                out_hbm.at[k, pl.ds(tbase + c * _CHT, _CHT)], ssem[p])
            if t + _NB < len(work):
                kn, cn = work[t + _NB]
                sca[p].wait()
                gat[p] = pltpu.async_copy(
                    table_hbm.at[cid, kn].at[idx_v.at[cn]], rows_v.at[p],
                    gsem[p])
                sca[p] = None
        for p in range(_NB):
            if sca[p] is not None:
                sca[p].wait()

    return sc_gather


def kernel(grip_state, distinction_table, state_table):
    B = grip_state.shape[0]
    num_kp, d_dist = distinction_table.shape
    d_state = state_table.shape[-1]
    info = plsc.get_sparse_core_info()
    NC, NS = info.num_cores, info.num_subcores

    g = grip_state.astype(jnp.int32).reshape(B // 8, 8)
    m = jnp.sum(g * (1 << jnp.arange(8, dtype=jnp.int32)), axis=1)
    m_sw = ((m & 15) << 4) | (m >> 4)

    out, _ = _build_sc_call(B, NC, NS, num_kp, d_dist, d_state)(
        distinction_table.reshape(-1),
        state_table.reshape(-1),
        m_sw)
    out = out.reshape(num_kp, B, d_dist + d_state)
    return jnp.transpose(out, (1, 0, 2))
